# baseline (speedup 1.0000x reference)
import functools

import jax
import jax.numpy as jnp
from jax import lax
from jax.experimental import pallas as pl
from jax.experimental.pallas import tpu as pltpu
from jax.experimental.pallas import tpu_sc as plsc

_NUM_CORES = 2
_NUM_SUBCORES = 16
_NUM_WORKERS = _NUM_CORES * _NUM_SUBCORES
_LANES = 128


@functools.cache
def _make_repack_kernel(V, D, dtype, rows_per_block=2048):
    r = _LANES // D
    n_blocks = (V + rows_per_block - 1) // rows_per_block
    out_rows_per_block = rows_per_block // r

    def body(tab_ref, out_ref):
        x = tab_ref[...]
        for s in range(r):
            out_ref[:, s * D:(s + 1) * D] = x[s::r, :]

    return pl.pallas_call(
        body,
        grid=(n_blocks,),
        in_specs=[pl.BlockSpec((rows_per_block, D), lambda i: (i, 0))],
        out_specs=pl.BlockSpec((out_rows_per_block, _LANES),
                               lambda i: (i, 0)),
        out_shape=jax.ShapeDtypeStruct((n_blocks * out_rows_per_block, _LANES),
                                       dtype),
    )


@functools.cache
def _make_gather_kernel(B, V_pack, dtype):
    b_per_w = B // _NUM_WORKERS
    ch = 128
    n_ch = b_per_w // ch
    mesh = plsc.VectorSubcoreMesh(core_axis_name="c", subcore_axis_name="s")
    out = jax.ShapeDtypeStruct((B, _LANES), dtype)

    @functools.partial(
        pl.kernel,
        mesh=mesh,
        out_type=(out, out),
        scratch_types=[
            pltpu.VMEM((b_per_w,), jnp.int32),
            pltpu.VMEM((b_per_w,), jnp.int32),
            pltpu.VMEM((ch, _LANES), dtype),
            pltpu.VMEM((ch, _LANES), dtype),
            pltpu.SemaphoreType.DMA,
            pltpu.SemaphoreType.DMA,
        ],
    )
    def k(user_tab, item_tab, u_idx, i_idx, u_out, i_out,
          uidx_v, iidx_v, urows_v, irows_v, usem, isem):
        wid = lax.axis_index("s") * _NUM_CORES + lax.axis_index("c")
        base = wid * b_per_w
        pltpu.sync_copy(u_idx.at[pl.ds(base, b_per_w)], uidx_v)
        pltpu.sync_copy(i_idx.at[pl.ds(base, b_per_w)], iidx_v)

        @pl.loop(0, n_ch)
        def _(c):
            cbase = c * ch
            ucp = pltpu.async_copy(
                user_tab.at[uidx_v.at[pl.ds(cbase, ch)]], urows_v, usem)
            icp = pltpu.async_copy(
                item_tab.at[iidx_v.at[pl.ds(cbase, ch)]], irows_v, isem)
            ucp.wait()
            pltpu.sync_copy(urows_v, u_out.at[pl.ds(base + cbase, ch)])
            icp.wait()
            pltpu.sync_copy(irows_v, i_out.at[pl.ds(base + cbase, ch)])

    return k


@functools.cache
def _make_extract_kernel(B, D, dtype, rows_per_block=2048):
    r = _LANES // D
    n_blocks = B // rows_per_block

    def body(rows_ref, sub_ref, out_ref):
        rows = rows_ref[...]
        sub = sub_ref[...]
        acc = rows[:, 0:D]
        for s in range(1, r):
            acc = jnp.where(sub == s, rows[:, s * D:(s + 1) * D], acc)
        out_ref[...] = acc

    return pl.pallas_call(
        body,
        grid=(n_blocks,),
        in_specs=[
            pl.BlockSpec((rows_per_block, _LANES), lambda i: (i, 0)),
            pl.BlockSpec((rows_per_block, D), lambda i: (i, 0)),
        ],
        out_specs=pl.BlockSpec((rows_per_block, D), lambda i: (i, 0)),
        out_shape=jax.ShapeDtypeStruct((B, D), dtype),
    )


@jax.jit
def kernel(user, item, user_table, item_table):
    B = user.shape[0]
    V, D = user_table.shape
    r = _LANES // D
    v_pad = ((V + r - 1) // r) * r
    u_pack = jnp.pad(user_table, ((0, v_pad - V), (0, 0))).reshape(
        v_pad // r, _LANES)
    i_pack = jnp.pad(item_table, ((0, v_pad - V), (0, 0))).reshape(
        v_pad // r, _LANES)
    user = user.astype(jnp.int32)
    item = item.astype(jnp.int32)
    u_phys = user // r
    i_phys = item // r
    u_sub = user % r
    i_sub = item % r
    k = _make_gather_kernel(B, u_pack.shape[0], user_table.dtype)
    u_rows, i_rows = k(u_pack, i_pack, u_phys, i_phys)
    extract = _make_extract_kernel(B, D, user_table.dtype)
    u_sub32 = jnp.broadcast_to(u_sub[:, None], (B, D))
    i_sub32 = jnp.broadcast_to(i_sub[:, None], (B, D))
    return extract(u_rows, u_sub32), extract(i_rows, i_sub32)
